# baseline (speedup 1.0000x reference)
import jax
import jax.numpy as jnp
from jax.experimental import pallas as pl
from jax.experimental.pallas import tpu as pltpu

_NBLK = 400
_K = 128


def _cost_body(logits_ref, kp_ref, tgt_ref, ids_ref, nb_ref, out_ref,
               a_ref, b_ref):
    inv = 1.0 / nb_ref[0, 0]
    nblk = logits_ref.shape[0]
    nc = logits_ref.shape[1]

    tgtT = tgt_ref[...].T
    ids = ids_ref[...]

    lg = logits_ref[...]
    m = jnp.max(lg, axis=1, keepdims=True)
    e = jnp.exp(lg - m)
    p = e / jnp.sum(e, axis=1, keepdims=True)

    w_vis = 0.2 * inv
    w_ctr = 0.5 * inv
    vp = kp_ref[:, 36:53]
    cp = kp_ref[:, 0:2]
    vg = tgtT[36:53, :]
    cg = tgtT[0:2, :]

    a_ref[...] = jnp.zeros(a_ref.shape, jnp.float32)
    a_ref[:, 0:17] = (-2.0 * w_vis) * vp
    a_ref[:, 17:19] = (-2.0 * w_ctr) * cp
    a_ref[:, 19:20] = (w_vis * jnp.sum(vp * vp, axis=1, keepdims=True)
                       + w_ctr * jnp.sum(cp * cp, axis=1, keepdims=True))
    a_ref[:, 20:21] = jnp.ones((nblk, 1), jnp.float32)
    a_ref[:, 21:21 + nc] = -p

    b_ref[...] = jnp.zeros(b_ref.shape, jnp.float32)
    b_ref[0:17, :] = vg
    b_ref[17:19, :] = cg
    b_ref[19:20, :] = jnp.ones((1, b_ref.shape[1]), jnp.float32)
    b_ref[20:21, :] = (w_vis * jnp.sum(vg * vg, axis=0, keepdims=True)
                       + w_ctr * jnp.sum(cg * cg, axis=0, keepdims=True))
    for c in range(nc):
        b_ref[21 + c : 22 + c, :] = (ids == c).astype(jnp.float32)

    acc = jnp.dot(a_ref[...], b_ref[...],
                  preferred_element_type=jnp.float32)

    a = 0.5 * inv
    zp = kp_ref[:, 2:36] * a
    zg = tgtT[2:36, :] * a
    cps = cp * a
    cgs = cg * a
    dcx = cps[:, 0:1] - cgs[0:1, :]
    dcy = cps[:, 1:2] - cgs[1:2, :]
    for k in range(17):
        d0, d1 = 2 * k, 2 * k + 1
        dz0 = zp[:, d0 : d0 + 1] - zg[d0 : d0 + 1, :]
        dz1 = zp[:, d1 : d1 + 1] - zg[d1 : d1 + 1, :]
        s1 = jnp.abs(dz0) + jnp.abs(dz1)
        s2 = jnp.abs(dz0 + dcx) + jnp.abs(dz1 + dcy)
        acc = acc + (s1 + 8.0 * s2) * vg[k : k + 1, :]
    out_ref[...] = acc


def kernel(pred_logits, pred_keypoints, tgt_ids, tgt_keypoints, num_boxes):
    bs, q, nc = pred_logits.shape
    n = bs * q
    t = tgt_keypoints.shape[0]

    logits2d = pred_logits.reshape(n, nc)
    kp2d = pred_keypoints.reshape(n, 53)
    ids_row = tgt_ids.reshape(1, t).astype(jnp.int32)
    nb = jnp.asarray(num_boxes, jnp.float32).reshape(1, 1)

    grid = (n // _NBLK,)
    out = pl.pallas_call(
        _cost_body,
        grid=grid,
        in_specs=[
            pl.BlockSpec((_NBLK, nc), lambda i: (i, 0)),
            pl.BlockSpec((_NBLK, 53), lambda i: (i, 0)),
            pl.BlockSpec((t, 53), lambda i: (0, 0)),
            pl.BlockSpec((1, t), lambda i: (0, 0)),
            pl.BlockSpec((1, 1), lambda i: (0, 0)),
        ],
        out_specs=pl.BlockSpec((_NBLK, t), lambda i: (i, 0)),
        out_shape=jax.ShapeDtypeStruct((n, t), jnp.float32),
        scratch_shapes=[
            pltpu.VMEM((_NBLK, _K), jnp.float32),
            pltpu.VMEM((_K, t), jnp.float32),
        ],
        compiler_params=pltpu.CompilerParams(
            dimension_semantics=("parallel",),
        ),
    )(logits2d, kp2d, tgt_keypoints, ids_row, nb)
    return out.reshape(bs, q, t)


# speedup vs baseline: 1.4623x; 1.4623x over previous
import jax
import jax.numpy as jnp
from jax.experimental import pallas as pl
from jax.experimental.pallas import tpu as pltpu

_K = 128


def _cost_body(logits_ref, kp_ref, tgt_ref, ids_ref, nb_ref, out_ref,
               a_ref, b_ref):
    inv = 1.0 / nb_ref[0, 0]
    q = logits_ref.shape[1]
    nc = logits_ref.shape[2]

    tgtT = tgt_ref[...].T
    ids = ids_ref[...]

    lg = logits_ref[0]
    m = jnp.max(lg, axis=1, keepdims=True)
    e = jnp.exp(lg - m)
    p = e / jnp.sum(e, axis=1, keepdims=True)

    w_vis = 0.2 * inv
    w_ctr = 0.5 * inv
    kp = kp_ref[0]
    vp = kp[:, 36:53]
    cp = kp[:, 0:2]
    vg = tgtT[36:53, :]
    cg = tgtT[0:2, :]

    a_ref[...] = jnp.zeros(a_ref.shape, jnp.float32)
    a_ref[:, 0:17] = (-2.0 * w_vis) * vp
    a_ref[:, 17:19] = (-2.0 * w_ctr) * cp
    a_ref[:, 19:20] = (w_vis * jnp.sum(vp * vp, axis=1, keepdims=True)
                       + w_ctr * jnp.sum(cp * cp, axis=1, keepdims=True))
    a_ref[:, 20:21] = jnp.ones((q, 1), jnp.float32)
    a_ref[:, 21:21 + nc] = -p

    b_ref[...] = jnp.zeros(b_ref.shape, jnp.float32)
    b_ref[0:17, :] = vg
    b_ref[17:19, :] = cg
    b_ref[19:20, :] = jnp.ones((1, b_ref.shape[1]), jnp.float32)
    b_ref[20:21, :] = (w_vis * jnp.sum(vg * vg, axis=0, keepdims=True)
                       + w_ctr * jnp.sum(cg * cg, axis=0, keepdims=True))
    for c in range(nc):
        b_ref[21 + c : 22 + c, :] = (ids == c).astype(jnp.float32)

    acc = jnp.dot(a_ref[...], b_ref[...],
                  preferred_element_type=jnp.float32)

    a = 0.5 * inv
    zp = kp[:, 2:36] * a
    zg = tgtT[2:36, :] * a
    cps = cp * a
    cgs = cg * a
    dcx = cps[:, 0:1] - cgs[0:1, :]
    dcy = cps[:, 1:2] - cgs[1:2, :]
    for k in range(17):
        d0, d1 = 2 * k, 2 * k + 1
        dz0 = zp[:, d0 : d0 + 1] - zg[d0 : d0 + 1, :]
        dz1 = zp[:, d1 : d1 + 1] - zg[d1 : d1 + 1, :]
        s1 = jnp.abs(dz0) + jnp.abs(dz1)
        s2 = jnp.abs(dz0 + dcx) + jnp.abs(dz1 + dcy)
        acc = acc + (s1 + 8.0 * s2) * vg[k : k + 1, :]
    out_ref[0] = acc


def kernel(pred_logits, pred_keypoints, tgt_ids, tgt_keypoints, num_boxes):
    bs, q, nc = pred_logits.shape
    t = tgt_keypoints.shape[0]

    ids_row = tgt_ids.reshape(1, t).astype(jnp.int32)
    nb = jnp.asarray(num_boxes, jnp.float32).reshape(1, 1)

    return pl.pallas_call(
        _cost_body,
        grid=(bs,),
        in_specs=[
            pl.BlockSpec((1, q, nc), lambda i: (i, 0, 0)),
            pl.BlockSpec((1, q, 53), lambda i: (i, 0, 0)),
            pl.BlockSpec((t, 53), lambda i: (0, 0)),
            pl.BlockSpec((1, t), lambda i: (0, 0)),
            pl.BlockSpec((1, 1), lambda i: (0, 0)),
        ],
        out_specs=pl.BlockSpec((1, q, t), lambda i: (i, 0, 0)),
        out_shape=jax.ShapeDtypeStruct((bs, q, t), jnp.float32),
        scratch_shapes=[
            pltpu.VMEM((q, _K), jnp.float32),
            pltpu.VMEM((_K, t), jnp.float32),
        ],
        compiler_params=pltpu.CompilerParams(
            dimension_semantics=("parallel",),
        ),
    )(pred_logits, pred_keypoints, tgt_keypoints, ids_row, nb)
